# baseline (speedup 1.0000x reference)
import functools

import jax
import jax.numpy as jnp
from jax.experimental import pallas as pl
from jax.experimental.pallas import tpu as pltpu

_N = 4096
_D = 256
_K = 10
_BR = 512
_NB = _N // _BR
_NSEG = 32
_T = 4


def _fused_kernel(t_ref, x_ref, w_ref, xp_ref, vals_ref, idx_ref,
                  xps_ref, sqf_ref, d2_ref):
    i = pl.program_id(0)

    @pl.when(i == 0)
    def _project():
        xp = jax.lax.dot_general(
            x_ref[...], w_ref[...], (((1,), (0,)), ((), ())),
            preferred_element_type=jnp.float32)
        xps_ref[...] = xp
        xp_ref[...] = xp
        sqf_ref[...] = jnp.sum(xp * xp, axis=1)[None, :]

    d2 = d2_ref[(i - 1) % 2]
    ip = jnp.minimum(i, _NB - 1)
    xb = xps_ref[pl.ds(ip * _BR, _BR), :]
    g2 = jax.lax.dot_general(
        xb * (-2.0), xps_ref[...], (((1,), (1,)), ((), ())),
        preferred_element_type=jnp.float32)
    sqb = jnp.sum(xb * xb, axis=1)[:, None]
    d2_ref[i % 2] = (sqb + sqf_ref[...]) + g2

    if True:
        inf = jnp.float32(jnp.inf)

        masked = [d2[:, j * 128:(j + 1) * 128] for j in range(_NSEG)]
        vs, js = [], []
        for r in range(_T):
            cur = masked[0]
            icur = jnp.zeros(cur.shape, jnp.int32)
            for j in range(1, _NSEG):
                ltm = masked[j] < cur
                cur = jnp.where(ltm, masked[j], cur)
                icur = jnp.where(ltm, j, icur)
            vs.append(cur)
            js.append(icur)
            if r + 1 < _T:
                masked = [jnp.where(mj == cur, inf, mj) for mj in masked]

        lane = jax.lax.broadcasted_iota(jnp.int32, vs[0].shape, 1)
        cur, c2, c3, c4 = vs
        icur, ic2, ic3, ic4 = js
        outv, outi = [], []
        for _ in range(_K):
            m = jnp.min(cur, axis=1)
            hit = cur == m[:, None]
            o = jnp.min(jnp.where(hit, lane, _N), axis=1)
            win = lane == o[:, None]
            j32 = jnp.min(jnp.where(win, icur, _NSEG), axis=1)
            outv.append(m)
            outi.append(j32 * 128 + o)
            cur = jnp.where(win, c2, cur)
            c2 = jnp.where(win, c3, c2)
            c3 = jnp.where(win, c4, c3)
            c4 = jnp.where(win, inf, c4)
            icur = jnp.where(win, ic2, icur)
            ic2 = jnp.where(win, ic3, ic2)
            ic3 = jnp.where(win, ic4, ic3)
        t = t_ref[0, 0]
        vals_ref[...] = jnp.stack(outv, axis=1) * (-t)
        idx_ref[...] = jnp.stack(outi, axis=1)


@functools.partial(jax.jit, static_argnames=())
def kernel(x, A, W, temperature):
    del A
    n, d = x.shape
    t = jnp.exp(jnp.clip(temperature, -5.0, 5.0)).reshape(1, 1)

    def _prev(i):
        return jnp.where(i == 0, 0, i - 1)

    xp, vals, idx = pl.pallas_call(
        _fused_kernel,
        grid=(_NB + 1,),
        in_specs=[
            pl.BlockSpec((1, 1), lambda i: (0, 0), memory_space=pltpu.SMEM),
            pl.BlockSpec((n, d), lambda i: (0, 0)),
            pl.BlockSpec((d, d), lambda i: (0, 0)),
        ],
        out_specs=[
            pl.BlockSpec((n, d), lambda i: (0, 0)),
            pl.BlockSpec((_BR, _K), lambda i: (_prev(i), 0)),
            pl.BlockSpec((_BR, _K), lambda i: (_prev(i), 0)),
        ],
        out_shape=[
            jax.ShapeDtypeStruct((n, d), jnp.float32),
            jax.ShapeDtypeStruct((n, _K), jnp.float32),
            jax.ShapeDtypeStruct((n, _K), jnp.int32),
        ],
        scratch_shapes=[
            pltpu.VMEM((n, d), jnp.float32),
            pltpu.VMEM((1, n), jnp.float32),
            pltpu.VMEM((2, _BR, n), jnp.float32),
        ],
    )(t, x, W)

    logprobs = vals[None]
    rows = jnp.repeat(jnp.arange(n, dtype=jnp.int32), _K)
    edges_hat = jnp.stack([idx.reshape(-1), rows], axis=0)
    return (xp[None], edges_hat, logprobs)


# speedup vs baseline: 1.2495x; 1.2495x over previous
import functools

import jax
import jax.numpy as jnp
from jax.experimental import pallas as pl
from jax.experimental.pallas import tpu as pltpu

_N = 4096
_D = 256
_K = 10
_BR = 512
_NSEG = 32
_T = 4


def _proj_kernel(x_ref, w_ref, xp_ref):
    xp_ref[...] = jax.lax.dot_general(
        x_ref[...], w_ref[...], (((1,), (0,)), ((), ())),
        preferred_element_type=jnp.float32)


def _dist_topk_kernel(t_ref, xpb_ref, xp_ref, vals_ref, idx_ref, sqf_ref):
    i = pl.program_id(0)
    xf = xp_ref[...]

    @pl.when(i == 0)
    def _colnorms():
        sqf_ref[...] = jnp.sum(xf * xf, axis=1)[None, :]

    xb = xpb_ref[...]
    g2 = jax.lax.dot_general(
        xb * (-2.0), xf, (((1,), (1,)), ((), ())),
        preferred_element_type=jnp.float32)
    sqb = jnp.sum(xb * xb, axis=1)[:, None]
    d2 = (sqb + sqf_ref[...]) + g2
    inf = jnp.float32(jnp.inf)

    masked = [d2[:, j * 128:(j + 1) * 128] for j in range(_NSEG)]
    vs, js = [], []
    for r in range(_T):
        cur = masked[0]
        icur = jnp.zeros(cur.shape, jnp.int32)
        for j in range(1, _NSEG):
            ltm = masked[j] < cur
            cur = jnp.where(ltm, masked[j], cur)
            icur = jnp.where(ltm, j, icur)
        vs.append(cur)
        js.append(icur)
        if r + 1 < _T:
            masked = [jnp.where(mj == cur, inf, mj) for mj in masked]

    lane = jax.lax.broadcasted_iota(jnp.int32, vs[0].shape, 1)
    cur, c2, c3, c4 = vs
    icur, ic2, ic3, ic4 = js
    outv, outi = [], []
    for _ in range(_K):
        m = jnp.min(cur, axis=1)
        hit = cur == m[:, None]
        o = jnp.min(jnp.where(hit, lane, _N), axis=1)
        win = lane == o[:, None]
        j32 = jnp.min(jnp.where(win, icur, _NSEG), axis=1)
        outv.append(m)
        outi.append(j32 * 128 + o)
        cur = jnp.where(win, c2, cur)
        c2 = jnp.where(win, c3, c2)
        c3 = jnp.where(win, c4, c3)
        c4 = jnp.where(win, inf, c4)
        icur = jnp.where(win, ic2, icur)
        ic2 = jnp.where(win, ic3, ic2)
        ic3 = jnp.where(win, ic4, ic3)
    t = t_ref[0, 0]
    vals_ref[...] = jnp.stack(outv, axis=1) * (-t)
    idx_ref[...] = jnp.stack(outi, axis=1)


@functools.partial(jax.jit, static_argnames=())
def kernel(x, A, W, temperature):
    del A
    n, d = x.shape
    t = jnp.exp(jnp.clip(temperature, -5.0, 5.0)).reshape(1, 1)

    xp = pl.pallas_call(
        _proj_kernel,
        out_shape=jax.ShapeDtypeStruct((n, d), jnp.float32),
    )(x, W)

    grid = (n // _BR,)
    vals, idx = pl.pallas_call(
        _dist_topk_kernel,
        grid=grid,
        in_specs=[
            pl.BlockSpec((1, 1), lambda i: (0, 0), memory_space=pltpu.SMEM),
            pl.BlockSpec((_BR, d), lambda i: (i, 0)),
            pl.BlockSpec((n, d), lambda i: (0, 0)),
        ],
        out_specs=[
            pl.BlockSpec((_BR, _K), lambda i: (i, 0)),
            pl.BlockSpec((_BR, _K), lambda i: (i, 0)),
        ],
        out_shape=[
            jax.ShapeDtypeStruct((n, _K), jnp.float32),
            jax.ShapeDtypeStruct((n, _K), jnp.int32),
        ],
        scratch_shapes=[
            pltpu.VMEM((1, n), jnp.float32),
        ],
    )(t, xp, xp)

    logprobs = vals[None]
    rows = jnp.repeat(jnp.arange(n, dtype=jnp.int32), _K)
    edges_hat = jnp.stack([idx.reshape(-1), rows], axis=0)
    return (xp[None], edges_hat, logprobs)


# speedup vs baseline: 1.2897x; 1.0322x over previous
import functools

import jax
import jax.numpy as jnp
from jax.experimental import pallas as pl
from jax.experimental.pallas import tpu as pltpu

_N = 4096
_D = 256
_K = 10
_BR = 512
_NSEG = 32
_T = 3


def _proj_kernel(x_ref, w_ref, xp_ref):
    xp_ref[...] = jax.lax.dot_general(
        x_ref[...], w_ref[...], (((1,), (0,)), ((), ())),
        preferred_element_type=jnp.float32)


def _dist_topk_kernel(t_ref, xpb_ref, xp_ref, vals_ref, idx_ref, sqf_ref):
    i = pl.program_id(0)
    xf = xp_ref[...]

    @pl.when(i == 0)
    def _colnorms():
        sqf_ref[...] = jnp.sum(xf * xf, axis=1)[None, :]

    xb = xpb_ref[...]
    g2 = jax.lax.dot_general(
        xb * (-2.0), xf, (((1,), (1,)), ((), ())),
        preferred_element_type=jnp.float32)
    sqb = jnp.sum(xb * xb, axis=1)[:, None]
    d2 = (sqb + sqf_ref[...]) + g2
    inf = jnp.float32(jnp.inf)

    segs = [d2[:, j * 128:(j + 1) * 128] for j in range(_NSEG)]
    half = _NSEG // 2
    vs, js = [], []
    fams = []
    for f in range(2):
        masked = segs[f * half:(f + 1) * half]
        fvs, fjs = [], []
        for r in range(_T):
            cur = masked[0]
            icur = jnp.full(cur.shape, f * half, jnp.int32)
            for j in range(1, half):
                ltm = masked[j] < cur
                cur = jnp.where(ltm, masked[j], cur)
                icur = jnp.where(ltm, f * half + j, icur)
            fvs.append(cur)
            fjs.append(icur)
            if r + 1 < _T:
                masked = [jnp.where(mj == cur, inf, mj) for mj in masked]
        fams.append((fvs, fjs))
    for r in range(_T):
        vs.append(jnp.concatenate([fams[0][0][r], fams[1][0][r]], axis=1))
        js.append(jnp.concatenate([fams[0][1][r], fams[1][1][r]], axis=1))

    lane = jax.lax.broadcasted_iota(jnp.int32, vs[0].shape, 1)
    cur, c2, c3 = vs
    icur, ic2, ic3 = js
    outv, outi = [], []
    for _ in range(_K):
        m = jnp.min(cur, axis=1)
        hit = cur == m[:, None]
        o = jnp.min(jnp.where(hit, lane, _N), axis=1)
        win = lane == o[:, None]
        j32 = jnp.min(jnp.where(win, icur, _NSEG), axis=1)
        outv.append(m)
        outi.append(j32 * 128 + (o & 127))
        cur = jnp.where(win, c2, cur)
        c2 = jnp.where(win, c3, c2)
        c3 = jnp.where(win, inf, c3)
        icur = jnp.where(win, ic2, icur)
        ic2 = jnp.where(win, ic3, ic2)
    t = t_ref[0, 0]
    vals_ref[...] = jnp.stack(outv, axis=1) * (-t)
    idx_ref[...] = jnp.stack(outi, axis=1)


@functools.partial(jax.jit, static_argnames=())
def kernel(x, A, W, temperature):
    del A
    n, d = x.shape
    t = jnp.exp(jnp.clip(temperature, -5.0, 5.0)).reshape(1, 1)

    xp = pl.pallas_call(
        _proj_kernel,
        out_shape=jax.ShapeDtypeStruct((n, d), jnp.float32),
    )(x, W)

    grid = (n // _BR,)
    vals, idx = pl.pallas_call(
        _dist_topk_kernel,
        grid=grid,
        in_specs=[
            pl.BlockSpec((1, 1), lambda i: (0, 0), memory_space=pltpu.SMEM),
            pl.BlockSpec((_BR, d), lambda i: (i, 0)),
            pl.BlockSpec((n, d), lambda i: (0, 0)),
        ],
        out_specs=[
            pl.BlockSpec((_BR, _K), lambda i: (i, 0)),
            pl.BlockSpec((_BR, _K), lambda i: (i, 0)),
        ],
        out_shape=[
            jax.ShapeDtypeStruct((n, _K), jnp.float32),
            jax.ShapeDtypeStruct((n, _K), jnp.int32),
        ],
        scratch_shapes=[
            pltpu.VMEM((1, n), jnp.float32),
        ],
    )(t, xp, xp)

    logprobs = vals[None]
    rows = jnp.repeat(jnp.arange(n, dtype=jnp.int32), _K)
    edges_hat = jnp.stack([idx.reshape(-1), rows], axis=0)
    return (xp[None], edges_hat, logprobs)
